# baseline (speedup 1.0000x reference)
import functools

import jax
import jax.numpy as jnp
from jax import lax
from jax.experimental import pallas as pl
from jax.experimental.pallas import tpu as pltpu
from jax.experimental.pallas import tpu_sc as plsc

OLD_CL = 16
RATIO = 0.2

SC_ROWS = 64
R_STAGE = 2
BH = 224
SR = 8
N_WORK = 32


def _tc_partial_kernel(x_ref, out_ref, vec_ref, *, n_img, n_j, c, w):
    i = pl.program_id(0)
    j = pl.program_id(1)
    n_new = c - OLD_CL
    nq = 2 * n_new + 1

    @pl.when(j == 0)
    def _init_vec():
        vec_ref[:, :, :] = jnp.zeros_like(vec_ref)

    @pl.when((i == 0) & (j == 0))
    def _init_out():
        out_ref[:, :] = jnp.zeros_like(out_ref)

    accs = [jnp.zeros((SR, w), dtype=jnp.float32) for _ in range(nq)]
    for s in range(0, x_ref.shape[2], SR):
        sl = pl.ds(s, SR)
        e0 = jnp.exp(x_ref[0, 0, sl, :])
        m = e0
        s_old = e0
        for ci in range(1, OLD_CL):
            e = jnp.exp(x_ref[0, ci, sl, :])
            m = jnp.maximum(m, e)
            s_old = s_old + e
        e_new = []
        z = s_old
        for ci in range(OLD_CL, c):
            e = jnp.exp(x_ref[0, ci, sl, :])
            m = jnp.maximum(m, e)
            z = z + e
            e_new.append(e)
        inv = 1.0 / z

        p0 = s_old * inv
        accs[0] = accs[0] + p0 * p0
        for k, e in enumerate(e_new):
            p = e * inv
            accs[1 + k] = accs[1 + k] + p * p
        for k, e in enumerate(e_new):
            accs[1 + n_new + k] = accs[1 + n_new + k] + jnp.where(
                e == m, 1.0, 0.0)

    for k in range(nq):
        vec_ref[k, :, :] = vec_ref[k, :, :] + accs[k]

    @pl.when(j == n_j - 1)
    def _flush():
        row = jax.lax.broadcasted_iota(jnp.int32, (8, 16), 0)
        lane = jax.lax.broadcasted_iota(jnp.int32, (8, 16), 1)
        upd = jnp.zeros((8, 16), dtype=jnp.float32)
        for k in range(nq):
            v = jnp.sum(vec_ref[k, :, :])
            upd = upd + jnp.where((row == i) & (lane == k), v, 0.0)
        out_ref[:, :] = out_ref[:, :] + upd


def _sc_partial_kernel(x_hbm, out_hbm, buf, vout, *, c, h, w):
    n_new = c - OLD_CL
    nq = 2 * n_new + 1
    cid = lax.axis_index("c")
    sid = lax.axis_index("s")
    wid = sid * 2 + cid
    img = wid // 8
    sub = wid % 8
    rows_per_worker = SC_ROWS // 8
    row0 = (h - SC_ROWS) + sub * rows_per_worker

    accs = tuple(jnp.zeros((16,), dtype=jnp.float32) for _ in range(nq))
    for chunk in range(rows_per_worker // R_STAGE):
        r0 = row0 + chunk * R_STAGE
        pltpu.sync_copy(x_hbm.at[img, :, pl.ds(r0, R_STAGE), :], buf)
        for rr in range(R_STAGE):
            def body(g, carry, _rr=rr):
                col = g * 16
                e0 = jnp.exp(buf[0, _rr, pl.ds(col, 16)])
                m = e0
                s_old = e0
                for ci in range(1, OLD_CL):
                    e = jnp.exp(buf[ci, _rr, pl.ds(col, 16)])
                    m = jnp.maximum(m, e)
                    s_old = s_old + e
                e_new = []
                z = s_old
                for ci in range(OLD_CL, c):
                    e = jnp.exp(buf[ci, _rr, pl.ds(col, 16)])
                    m = jnp.maximum(m, e)
                    z = z + e
                    e_new.append(e)
                inv = 1.0 / z
                p0 = s_old * inv
                out = [carry[0] + p0 * p0]
                for k, e in enumerate(e_new):
                    p = e * inv
                    out.append(carry[1 + k] + p * p)
                for k, e in enumerate(e_new):
                    out.append(carry[1 + n_new + k]
                               + jnp.where(e == m, 1.0, 0.0))
                return tuple(out)
            accs = lax.fori_loop(0, w // 16, body, accs)

    for k in range(nq):
        vout[k, :] = accs[k]
    pltpu.sync_copy(vout, out_hbm.at[wid])


def _merge_kernel(tc_ref, sc_ref, o_ref, *, n_img, c, h, w):
    n_new = c - OLD_CL
    nbin = n_new + 1
    sc = sc_ref[:, :, :].sum(axis=2)
    sc4 = sc.reshape(n_img, N_WORK // n_img, 16).sum(axis=1)
    tab = tc_ref[:, :] + jnp.concatenate(
        [sc4, jnp.zeros((8 - n_img, 16), dtype=jnp.float32)], axis=0)
    row = jax.lax.broadcasted_iota(jnp.int32, (8, 16), 0)
    lane = jax.lax.broadcasted_iota(jnp.int32, (8, 16), 1)
    valid = row < n_img
    cnt_lane = (lane >= nbin) & (lane < nbin + n_new)
    s_new = jnp.sum(jnp.where(valid & cnt_lane, tab, 0.0),
                    axis=1, keepdims=True)
    bins = jnp.roll(tab, -n_new, axis=1)
    bins = jnp.where(lane == 0, float(h * w) - s_new, bins)
    bin_lane = lane < nbin
    safe = jnp.where(valid & bin_lane,
                     jnp.where(bins == 0.0, 1.0, bins), 1.0)
    total = jnp.sum(jnp.where(valid & bin_lane, safe, 0.0),
                    axis=1, keepdims=True)
    wgt = jnp.where(valid & bin_lane, jnp.power(total / safe, RATIO), 0.0)
    sq = jnp.where(valid & bin_lane, tab, 0.0)
    o_ref[0, 0] = -jnp.sum(sq * wgt) / (n_img * c * h * w)


def kernel(inputs):
    n, c, h, w = inputs.shape
    h_tc = h - SC_ROWS
    n_j = h_tc // BH
    nq = 2 * (c - OLD_CL) + 1

    tc_tab = pl.pallas_call(
        functools.partial(_tc_partial_kernel, n_img=n, n_j=n_j, c=c, w=w),
        grid=(n, n_j),
        in_specs=[pl.BlockSpec((1, c, BH, w), lambda i, j: (i, 0, j, 0))],
        out_specs=pl.BlockSpec((8, 16), lambda i, j: (0, 0)),
        out_shape=jax.ShapeDtypeStruct((8, 16), jnp.float32),
        scratch_shapes=[pltpu.VMEM((nq, SR, w), jnp.float32)],
    )(inputs)

    mesh = plsc.VectorSubcoreMesh(core_axis_name="c", subcore_axis_name="s")
    sc_tab = pl.kernel(
        functools.partial(_sc_partial_kernel, c=c, h=h, w=w),
        mesh=mesh,
        out_type=jax.ShapeDtypeStruct((N_WORK, 16, 16), jnp.float32),
        scratch_types=[
            pltpu.VMEM((c, R_STAGE, w), jnp.float32),
            pltpu.VMEM((16, 16), jnp.float32),
        ],
    )(inputs)

    out = pl.pallas_call(
        functools.partial(_merge_kernel, n_img=n, c=c, h=h, w=w),
        in_specs=[
            pl.BlockSpec((8, 16), lambda: (0, 0)),
            pl.BlockSpec((N_WORK, 16, 16), lambda: (0, 0, 0)),
        ],
        out_specs=pl.BlockSpec((1, 1), lambda: (0, 0),
                               memory_space=pltpu.SMEM),
        out_shape=jax.ShapeDtypeStruct((1, 1), jnp.float32),
    )(tc_tab, sc_tab)
    return out[0, 0]


# speedup vs baseline: 1.0196x; 1.0196x over previous
import functools

import jax
import jax.numpy as jnp
from jax import lax
from jax.experimental import pallas as pl
from jax.experimental.pallas import tpu as pltpu
from jax.experimental.pallas import tpu_sc as plsc

OLD_CL = 16
RATIO = 0.2

SC_ROWS = 64
R_STAGE = 2
BH = 224
SR = 8
N_WORK = 32


def _tc_partial_kernel(x_ref, out_ref, vec_ref, *, n_img, n_j, c, w):
    i = pl.program_id(0)
    j = pl.program_id(1)
    n_new = c - OLD_CL
    nq = 2 * n_new + 1

    @pl.when(j == 0)
    def _init_vec():
        vec_ref[:, :, :] = jnp.zeros_like(vec_ref)

    @pl.when((i == 0) & (j == 0))
    def _init_out():
        out_ref[:, :] = jnp.zeros_like(out_ref)

    accs = [jnp.zeros((SR, w), dtype=jnp.float32) for _ in range(nq)]
    for s in range(0, x_ref.shape[2], SR):
        sl = pl.ds(s, SR)
        e0 = jnp.exp(x_ref[0, 0, sl, :])
        m = e0
        s_old = e0
        for ci in range(1, OLD_CL):
            e = jnp.exp(x_ref[0, ci, sl, :])
            m = jnp.maximum(m, e)
            s_old = s_old + e
        e_new = []
        z = s_old
        for ci in range(OLD_CL, c):
            e = jnp.exp(x_ref[0, ci, sl, :])
            m = jnp.maximum(m, e)
            z = z + e
            e_new.append(e)
        inv = 1.0 / z

        p0 = s_old * inv
        accs[0] = accs[0] + p0 * p0
        for k, e in enumerate(e_new):
            p = e * inv
            accs[1 + k] = accs[1 + k] + p * p
        for k, e in enumerate(e_new):
            accs[1 + n_new + k] = accs[1 + n_new + k] + jnp.where(
                e == m, 1.0, 0.0)

    for k in range(nq):
        vec_ref[k, :, :] = vec_ref[k, :, :] + accs[k]

    @pl.when(j == n_j - 1)
    def _flush():
        row = jax.lax.broadcasted_iota(jnp.int32, (8, 16), 0)
        lane = jax.lax.broadcasted_iota(jnp.int32, (8, 16), 1)
        upd = jnp.zeros((8, 16), dtype=jnp.float32)
        for k in range(nq):
            v = jnp.sum(vec_ref[k, :, :])
            upd = upd + jnp.where((row == i) & (lane == k), v, 0.0)
        out_ref[:, :] = out_ref[:, :] + upd


def _sc_partial_kernel(x_hbm, out_hbm, buf, vout, *, c, h, w):
    n_new = c - OLD_CL
    nq = 2 * n_new + 1
    cid = lax.axis_index("c")
    sid = lax.axis_index("s")
    wid = sid * 2 + cid
    img = wid // 8
    sub = wid % 8
    rows_per_worker = SC_ROWS // 8
    row0 = (h - SC_ROWS) + sub * rows_per_worker

    accs = tuple(jnp.zeros((16,), dtype=jnp.float32) for _ in range(nq))
    for chunk in range(rows_per_worker // R_STAGE):
        r0 = row0 + chunk * R_STAGE
        pltpu.sync_copy(x_hbm.at[img, :, pl.ds(r0, R_STAGE), :], buf)
        for rr in range(R_STAGE):
            def body(g, carry, _rr=rr):
                col = g * 16
                e0 = jnp.exp(buf[0, _rr, pl.ds(col, 16)])
                m = e0
                s_old = e0
                for ci in range(1, OLD_CL):
                    e = jnp.exp(buf[ci, _rr, pl.ds(col, 16)])
                    m = jnp.maximum(m, e)
                    s_old = s_old + e
                e_new = []
                z = s_old
                for ci in range(OLD_CL, c):
                    e = jnp.exp(buf[ci, _rr, pl.ds(col, 16)])
                    m = jnp.maximum(m, e)
                    z = z + e
                    e_new.append(e)
                inv = 1.0 / z
                p0 = s_old * inv
                out = [carry[0] + p0 * p0]
                for k, e in enumerate(e_new):
                    p = e * inv
                    out.append(carry[1 + k] + p * p)
                for k, e in enumerate(e_new):
                    out.append(carry[1 + n_new + k]
                               + jnp.where(e == m, 1.0, 0.0))
                return tuple(out)
            accs = lax.fori_loop(0, w // 16, body, accs)

    for k in range(nq):
        vout[k, :] = accs[k]
    pltpu.sync_copy(vout, out_hbm.at[wid])


def _merge_kernel(tc_ref, sc_ref, o_ref, *, n_img, c, h, w):
    n_new = c - OLD_CL
    nbin = n_new + 1
    sc = sc_ref[:, :, :].sum(axis=2)
    sc4 = sc.reshape(n_img, N_WORK // n_img, 16).sum(axis=1)
    tab = tc_ref[:, :] + jnp.concatenate(
        [sc4, jnp.zeros((8 - n_img, 16), dtype=jnp.float32)], axis=0)
    row = jax.lax.broadcasted_iota(jnp.int32, (8, 16), 0)
    lane = jax.lax.broadcasted_iota(jnp.int32, (8, 16), 1)
    valid = row < n_img
    cnt_lane = (lane >= nbin) & (lane < nbin + n_new)
    s_new = jnp.sum(jnp.where(valid & cnt_lane, tab, 0.0),
                    axis=1, keepdims=True)
    bins = jnp.roll(tab, -n_new, axis=1)
    bins = jnp.where(lane == 0, float(h * w) - s_new, bins)
    bin_lane = lane < nbin
    safe = jnp.where(valid & bin_lane,
                     jnp.where(bins == 0.0, 1.0, bins), 1.0)
    total = jnp.sum(jnp.where(valid & bin_lane, safe, 0.0),
                    axis=1, keepdims=True)
    wgt = jnp.where(valid & bin_lane, jnp.power(total / safe, RATIO), 0.0)
    sq = jnp.where(valid & bin_lane, tab, 0.0)
    o_ref[0, 0] = -jnp.sum(sq * wgt) / (n_img * c * h * w)


def kernel(inputs):
    n, c, h, w = inputs.shape
    h_tc = h - SC_ROWS
    n_j = h_tc // BH
    nq = 2 * (c - OLD_CL) + 1

    mesh = plsc.VectorSubcoreMesh(core_axis_name="c", subcore_axis_name="s")
    sc_tab = pl.kernel(
        functools.partial(_sc_partial_kernel, c=c, h=h, w=w),
        mesh=mesh,
        out_type=jax.ShapeDtypeStruct((N_WORK, 16, 16), jnp.float32),
        scratch_types=[
            pltpu.VMEM((c, R_STAGE, w), jnp.float32),
            pltpu.VMEM((16, 16), jnp.float32),
        ],
    )(inputs)

    tc_tab = pl.pallas_call(
        functools.partial(_tc_partial_kernel, n_img=n, n_j=n_j, c=c, w=w),
        grid=(n, n_j),
        in_specs=[pl.BlockSpec((1, c, BH, w), lambda i, j: (i, 0, j, 0))],
        out_specs=pl.BlockSpec((8, 16), lambda i, j: (0, 0)),
        out_shape=jax.ShapeDtypeStruct((8, 16), jnp.float32),
        scratch_shapes=[pltpu.VMEM((nq, SR, w), jnp.float32)],
    )(inputs)

    out = pl.pallas_call(
        functools.partial(_merge_kernel, n_img=n, c=c, h=h, w=w),
        in_specs=[
            pl.BlockSpec((8, 16), lambda: (0, 0)),
            pl.BlockSpec((N_WORK, 16, 16), lambda: (0, 0, 0)),
        ],
        out_specs=pl.BlockSpec((1, 1), lambda: (0, 0),
                               memory_space=pltpu.SMEM),
        out_shape=jax.ShapeDtypeStruct((1, 1), jnp.float32),
    )(tc_tab, sc_tab)
    return out[0, 0]


# speedup vs baseline: 1.4068x; 1.3797x over previous
import functools

import jax
import jax.numpy as jnp
from jax.experimental import pallas as pl
from jax.experimental.pallas import tpu as pltpu

OLD_CL = 16
RATIO = 0.2
BH = 256


SR = 16


def _loss_kernel(x_ref, out_ref, vec_ref, img_ref, *, n_img, n_j, c, h, w):
    i = pl.program_id(0)
    j = pl.program_id(1)
    n_new = c - OLD_CL
    nq = 2 * n_new + 1

    @pl.when(j == 0)
    def _init_vec():
        vec_ref[:, :, :] = jnp.zeros_like(vec_ref)

    @pl.when((i == 0) & (j == 0))
    def _init_img():
        img_ref[:, :] = jnp.zeros_like(img_ref)

    accs = [jnp.zeros((SR, w), dtype=jnp.float32) for _ in range(nq)]
    for s in range(0, x_ref.shape[2], SR):
        sl = pl.ds(s, SR)
        e0 = jnp.exp(x_ref[0, 0, sl, :])
        m = e0
        s_old = e0
        for ci in range(1, OLD_CL):
            e = jnp.exp(x_ref[0, ci, sl, :])
            m = jnp.maximum(m, e)
            s_old = s_old + e
        e_new = []
        z = s_old
        for ci in range(OLD_CL, c):
            e = jnp.exp(x_ref[0, ci, sl, :])
            m = jnp.maximum(m, e)
            z = z + e
            e_new.append(e)
        inv = 1.0 / z

        p0 = s_old * inv
        accs[0] = accs[0] + p0 * p0
        for k, e in enumerate(e_new):
            p = e * inv
            accs[1 + k] = accs[1 + k] + p * p
        for k, e in enumerate(e_new):
            accs[1 + n_new + k] = accs[1 + n_new + k] + jnp.where(
                e == m, 1.0, 0.0)

    for k in range(nq):
        vec_ref[k, :, :] = vec_ref[k, :, :] + accs[k]

    @pl.when(j == n_j - 1)
    def _flush():
        row = jax.lax.broadcasted_iota(jnp.int32, (8, 128), 0)
        lane = jax.lax.broadcasted_iota(jnp.int32, (8, 128), 1)
        scalars = [jnp.sum(vec_ref[k, :, :]) for k in range(nq)]
        cnt_new = scalars[1 + n_new:]
        cnt_old = float(h * w) - sum(cnt_new)
        vals = scalars[: 1 + n_new] + [cnt_old] + cnt_new
        acc = img_ref[:, :]
        for k, v in enumerate(vals):
            acc = acc + jnp.where((row == i) & (lane == k), v, 0.0)
        img_ref[:, :] = acc

        @pl.when(i == n_img - 1)
        def _finish():
            nbin = n_new + 1
            sq_lane = lane < nbin
            cnt_lane = (lane >= nbin) & (lane < 2 * nbin)
            valid = row < n_img
            a = img_ref[:, :]
            cnt = jnp.where(valid & cnt_lane, a, 0.0)
            safe = jnp.where(valid & cnt_lane,
                             jnp.where(cnt == 0.0, 1.0, cnt), 1.0)
            total = jnp.sum(jnp.where(valid & cnt_lane, safe, 0.0),
                            axis=1, keepdims=True)
            wgt = jnp.where(valid & cnt_lane,
                            jnp.power(total / safe, RATIO), 0.0)
            sq = jnp.where(valid & sq_lane, a, 0.0)
            contrib = jnp.sum(sq * jnp.roll(wgt, -nbin, axis=1))
            out_ref[0, 0] = -contrib / (n_img * c * h * w)


def kernel(inputs):
    n, c, h, w = inputs.shape
    n_j = h // BH
    nq = 2 * (c - OLD_CL) + 1
    out = pl.pallas_call(
        functools.partial(_loss_kernel, n_img=n, n_j=n_j, c=c, h=h, w=w),
        grid=(n, n_j),
        in_specs=[
            pl.BlockSpec((1, c, BH, w), lambda i, j: (i, 0, j, 0)),
        ],
        out_specs=pl.BlockSpec(
            (1, 1), lambda i, j: (0, 0), memory_space=pltpu.SMEM
        ),
        out_shape=jax.ShapeDtypeStruct((1, 1), jnp.float32),
        scratch_shapes=[
            pltpu.VMEM((nq, SR, w), jnp.float32),
            pltpu.VMEM((8, 128), jnp.float32),
        ],
    )(inputs)
    return out[0, 0]


# speedup vs baseline: 1.5580x; 1.1075x over previous
import functools

import jax
import jax.numpy as jnp
from jax.experimental import pallas as pl
from jax.experimental.pallas import tpu as pltpu

OLD_CL = 16
RATIO = 0.2
BH = 256


SR = 8


def _loss_kernel(x_ref, out_ref, vec_ref, img_ref, *, n_img, n_j, c, h, w):
    i = pl.program_id(0)
    j = pl.program_id(1)
    n_new = c - OLD_CL
    nq = 2 * n_new + 1

    @pl.when(j == 0)
    def _init_vec():
        vec_ref[:, :, :] = jnp.zeros_like(vec_ref)

    @pl.when((i == 0) & (j == 0))
    def _init_img():
        img_ref[:, :] = jnp.zeros_like(img_ref)

    accs = [jnp.zeros((SR, w), dtype=jnp.float32) for _ in range(nq)]
    for s in range(0, x_ref.shape[2], SR):
        sl = pl.ds(s, SR)
        e0 = jnp.exp(x_ref[0, 0, sl, :])
        m = e0
        s_old = e0
        for ci in range(1, OLD_CL):
            e = jnp.exp(x_ref[0, ci, sl, :])
            m = jnp.maximum(m, e)
            s_old = s_old + e
        e_new = []
        z = s_old
        for ci in range(OLD_CL, c):
            e = jnp.exp(x_ref[0, ci, sl, :])
            m = jnp.maximum(m, e)
            z = z + e
            e_new.append(e)
        inv = 1.0 / z

        p0 = s_old * inv
        accs[0] = accs[0] + p0 * p0
        for k, e in enumerate(e_new):
            p = e * inv
            accs[1 + k] = accs[1 + k] + p * p
        for k, e in enumerate(e_new):
            accs[1 + n_new + k] = accs[1 + n_new + k] + jnp.where(
                e == m, 1.0, 0.0)

    for k in range(nq):
        vec_ref[k, :, :] = vec_ref[k, :, :] + accs[k]

    @pl.when(j == n_j - 1)
    def _flush():
        row = jax.lax.broadcasted_iota(jnp.int32, (8, 128), 0)
        lane = jax.lax.broadcasted_iota(jnp.int32, (8, 128), 1)
        scalars = [jnp.sum(vec_ref[k, :, :]) for k in range(nq)]
        cnt_new = scalars[1 + n_new:]
        cnt_old = float(h * w) - sum(cnt_new)
        vals = scalars[: 1 + n_new] + [cnt_old] + cnt_new
        acc = img_ref[:, :]
        for k, v in enumerate(vals):
            acc = acc + jnp.where((row == i) & (lane == k), v, 0.0)
        img_ref[:, :] = acc

        @pl.when(i == n_img - 1)
        def _finish():
            nbin = n_new + 1
            sq_lane = lane < nbin
            cnt_lane = (lane >= nbin) & (lane < 2 * nbin)
            valid = row < n_img
            a = img_ref[:, :]
            cnt = jnp.where(valid & cnt_lane, a, 0.0)
            safe = jnp.where(valid & cnt_lane,
                             jnp.where(cnt == 0.0, 1.0, cnt), 1.0)
            total = jnp.sum(jnp.where(valid & cnt_lane, safe, 0.0),
                            axis=1, keepdims=True)
            wgt = jnp.where(valid & cnt_lane,
                            jnp.power(total / safe, RATIO), 0.0)
            sq = jnp.where(valid & sq_lane, a, 0.0)
            contrib = jnp.sum(sq * jnp.roll(wgt, -nbin, axis=1))
            out_ref[0, 0] = -contrib / (n_img * c * h * w)


def kernel(inputs):
    n, c, h, w = inputs.shape
    n_j = h // BH
    nq = 2 * (c - OLD_CL) + 1
    out = pl.pallas_call(
        functools.partial(_loss_kernel, n_img=n, n_j=n_j, c=c, h=h, w=w),
        grid=(n, n_j),
        in_specs=[
            pl.BlockSpec((1, c, BH, w), lambda i, j: (i, 0, j, 0)),
        ],
        out_specs=pl.BlockSpec(
            (1, 1), lambda i, j: (0, 0), memory_space=pltpu.SMEM
        ),
        out_shape=jax.ShapeDtypeStruct((1, 1), jnp.float32),
        scratch_shapes=[
            pltpu.VMEM((nq, 8, w), jnp.float32),
            pltpu.VMEM((8, 128), jnp.float32),
        ],
    )(inputs)
    return out[0, 0]
